# Initial kernel scaffold; baseline (speedup 1.0000x reference)
import jax
import jax.numpy as jnp
from jax.experimental import pallas as pl


def kernel(x, conv_w, conv_b, bn_gamma, bn_beta, bn_mean, bn_var, head_w, head_b):
    raise NotImplementedError("write your pallas kernel here")


# speedup vs baseline: 1.2508x; 1.2508x over previous
import jax
import jax.numpy as jnp
from jax.experimental import pallas as pl
from jax.experimental.pallas import tpu as pltpu

_EPS = 1e-5
_B, _C, _H, _W = 32, 3, 400, 600
_KS = 5
_HO, _WO, _OC = _H // _KS, _W // _KS, 16
_ROWS = _HO * _WO
_TM = 2400
_NSUB = _ROWS // _TM


def _fused_kernel(p_ref, w_ref, b_ref, wh_ref, hb_ref, o_ref):
    def body(t, carry):
        s0, s1 = carry
        rows = pl.ds(t * _TM, _TM)
        y = jnp.dot(p_ref[rows, :], w_ref[...],
                    preferred_element_type=jnp.float32)
        y = jnp.maximum(y + b_ref[...], 0.0)
        p0 = jnp.sum(y * wh_ref[0, rows, :], axis=0, keepdims=True)
        p1 = jnp.sum(y * wh_ref[1, rows, :], axis=0, keepdims=True)
        return (s0 + p0, s1 + p1)

    zero = jnp.zeros((1, _OC), jnp.float32)
    s0, s1 = jax.lax.fori_loop(0, _NSUB, body, (zero, zero))
    t0 = jnp.sum(s0) + hb_ref[0, 0]
    t1 = jnp.sum(s1) + hb_ref[0, 1]
    lane = jax.lax.broadcasted_iota(jnp.int32, (1, 1, 128), 2)
    o_ref[...] = jnp.where(lane == 0, t0, jnp.where(lane == 1, t1, 0.0))


def kernel(x, conv_w, conv_b, bn_gamma, bn_beta, bn_mean, bn_var,
           head_w, head_b):
    p = x.reshape(_B, _C, _HO, _KS, _WO, _KS)
    p = p.transpose(0, 2, 4, 1, 3, 5).reshape(_B * _ROWS, _C * _KS * _KS)

    bn_scale = bn_gamma * jax.lax.rsqrt(bn_var + _EPS)
    w_eff = conv_w.reshape(_OC, _C * _KS * _KS).T * bn_scale[None, :]
    b_eff = (bn_scale * (conv_b - bn_mean) + bn_beta).reshape(1, _OC)

    wh = head_w.reshape(2, _OC, _ROWS).transpose(0, 2, 1)

    hb = jnp.zeros((1, 128), jnp.float32).at[0, :2].set(head_b)

    out_pad = pl.pallas_call(
        _fused_kernel,
        out_shape=jax.ShapeDtypeStruct((_B, 1, 128), jnp.float32),
        grid_spec=pltpu.PrefetchScalarGridSpec(
            num_scalar_prefetch=0,
            grid=(_B,),
            in_specs=[
                pl.BlockSpec((_ROWS, _C * _KS * _KS), lambda i: (i, 0)),
                pl.BlockSpec((_C * _KS * _KS, _OC), lambda i: (0, 0)),
                pl.BlockSpec((1, _OC), lambda i: (0, 0)),
                pl.BlockSpec((2, _ROWS, _OC), lambda i: (0, 0, 0)),
                pl.BlockSpec((1, 128), lambda i: (0, 0)),
            ],
            out_specs=pl.BlockSpec((1, 1, 128), lambda i: (i, 0, 0)),
        ),
        compiler_params=pltpu.CompilerParams(
            dimension_semantics=("parallel",)),
    )(p, w_eff, b_eff, wh, hb)

    return out_pad[:, 0, :2]


# speedup vs baseline: 2.8395x; 2.2701x over previous
import jax
import jax.numpy as jnp
from jax.experimental import pallas as pl
from jax.experimental.pallas import tpu as pltpu

_EPS = 1e-5
_B, _C, _H, _W = 32, 3, 400, 600
_KS = 5
_HO, _WO, _OC = _H // _KS, _W // _KS, 16
_HT = 16
_RT = _HT * _KS
_NHT = _HO // _HT


def _fused_kernel(x_ref, wr_ref, s_ref, b_ref, whz_ref, o_ref):
    def tile_body(ht, carry):
        a0, a1 = carry
        r0 = ht * _RT
        h0 = ht * _HT
        for oc in range(_OC):
            acc = x_ref[0, 0, pl.ds(r0, _RT), :] * wr_ref[oc, 0]
            for c in range(1, _C):
                acc = acc + x_ref[0, c, pl.ds(r0, _RT), :] * wr_ref[oc, c]
            rows = jnp.dot(s_ref[...], acc,
                           preferred_element_type=jnp.float32)
            s = rows
            for sh in range(1, _KS):
                s = s + pltpu.roll(rows, _W - sh, 1)
            r = jnp.maximum(s + b_ref[oc], 0.0)
            a0 = a0 + jnp.sum(r * whz_ref[0, oc, pl.ds(h0, _HT), :],
                              axis=0, keepdims=True)
            a1 = a1 + jnp.sum(r * whz_ref[1, oc, pl.ds(h0, _HT), :],
                              axis=0, keepdims=True)
        return (a0, a1)

    zero = jnp.zeros((1, _W), jnp.float32)
    a0, a1 = jax.lax.fori_loop(0, _NHT, tile_body, (zero, zero))
    t0 = jnp.sum(a0)
    t1 = jnp.sum(a1)
    lane = jax.lax.broadcasted_iota(jnp.int32, (1, 1, 128), 2)
    o_ref[...] = jnp.where(lane == 0, t0, jnp.where(lane == 1, t1, 0.0))


def kernel(x, conv_w, conv_b, bn_gamma, bn_beta, bn_mean, bn_var,
           head_w, head_b):
    bn_scale = bn_gamma * jax.lax.rsqrt(bn_var + _EPS)
    w_sc = conv_w * bn_scale[:, None, None, None]
    b_eff = bn_scale * (conv_b - bn_mean) + bn_beta

    wr = jnp.tile(w_sc, (1, 1, _RT // _KS, _WO))

    row = jax.lax.broadcasted_iota(jnp.int32, (_HT, _RT), 0)
    col = jax.lax.broadcasted_iota(jnp.int32, (_HT, _RT), 1)
    s_mat = ((col >= _KS * row) & (col < _KS * row + _KS)).astype(jnp.float32)

    wh = head_w.reshape(2, _OC, _HO, _WO)
    whz = jnp.zeros((2, _OC, _HO, _W), jnp.float32)
    whz = whz.at[:, :, :, ::_KS].set(wh)

    out_pad = pl.pallas_call(
        _fused_kernel,
        out_shape=jax.ShapeDtypeStruct((_B, 1, 128), jnp.float32),
        grid_spec=pltpu.PrefetchScalarGridSpec(
            num_scalar_prefetch=0,
            grid=(_B,),
            in_specs=[
                pl.BlockSpec((1, _C, _H, _W), lambda b: (b, 0, 0, 0)),
                pl.BlockSpec((_OC, _C, _RT, _W), lambda b: (0, 0, 0, 0)),
                pl.BlockSpec((_HT, _RT), lambda b: (0, 0)),
                pl.BlockSpec(memory_space=pltpu.SMEM),
                pl.BlockSpec((2, _OC, _HO, _W), lambda b: (0, 0, 0, 0)),
            ],
            out_specs=pl.BlockSpec((1, 1, 128), lambda b: (b, 0, 0)),
        ),
        compiler_params=pltpu.CompilerParams(
            dimension_semantics=("parallel",)),
    )(x, wr, s_mat, b_eff, whz)

    return out_pad[:, 0, :2] + head_b[None, :]
